# baseline (speedup 1.0000x reference)
import functools

import jax
import jax.numpy as jnp
from jax import lax
from jax.experimental import pallas as pl
from jax.experimental.pallas import tpu as pltpu
from jax.experimental.pallas import tpu_sc as plsc

_B = 16
_SEQ = 2048
_HALF = 1024
_D = 128
_ROWS = _B * _SEQ
_NC = 2
_NS = 16
_NW = _NC * _NS
_PBAND = 128
_NBAND = _SEQ // _PBAND
_BB = _B // 2
_L = 16

_sc_mesh = plsc.VectorSubcoreMesh(core_axis_name="c", subcore_axis_name="s")


@functools.partial(
    pl.kernel,
    out_type=jax.ShapeDtypeStruct((_ROWS, _D), jnp.float32),
    mesh=_sc_mesh,
    scratch_types=[
        pltpu.VMEM((_BB, _PBAND), jnp.int32),
        pltpu.VMEM((2, _D), jnp.float32),
        pltpu.VMEM((_PBAND, _D), jnp.float32),
        pltpu.VMEM((_PBAND, _D), jnp.float32),
        pltpu.VMEM((_PBAND, _D), jnp.float32),
        pltpu.VMEM((_PBAND, _D), jnp.float32),
        pltpu.VMEM((_PBAND, _D), jnp.float32),
        pltpu.VMEM((_PBAND, _D), jnp.float32),
        pltpu.VMEM_SHARED((_NS * _PBAND, _D), jnp.float32),
        pltpu.SemaphoreType.DMA,
        pltpu.SemaphoreType.DMA,
        pltpu.SemaphoreType.DMA,
        pltpu.SemaphoreType.DMA,
        pltpu.SemaphoreType.DMA,
        pltpu.SemaphoreType.DMA,
        pltpu.SemaphoreType.DMA,
        pltpu.SemaphoreType.DMA,
        pltpu.SemaphoreType.DMA,
        pltpu.SemaphoreType.DMA,
        pltpu.SemaphoreType.DMA,
        pltpu.SemaphoreType.DMA,
        pltpu.SemaphoreType.DMA,
        pltpu.SemaphoreType.DMA,
        pltpu.SemaphoreType.DMA,
        pltpu.SemaphoreType.DMA,
        pltpu.SemaphoreType.DMA,
        pltpu.SemaphoreType.DMA,
        pltpu.SemaphoreType.DMA,
    ],
)
def _sc_embed(x_hbm, tok_hbm, pos_hbm, seg_hbm, out_hbm,
              idx_v, seg_v,
              rows0, rows1, rows2, rows3, rows4, rows5, comb_sp,
              sx, sc0, sc1, sc2, sc3, sc4, sc5,
              sg0, sg1, sg2, sg3, sg4, sg5,
              so0, so1, so2, so3, so4, so5):
    sid = lax.axis_index("s")
    wid = sid * _NC + lax.axis_index("c")
    band = wid // 2
    b0 = (wid % 2) * _BB
    p0 = band * _PBAND
    rows = (rows0, rows1, rows2, rows3, rows4, rows5)
    sem_c = (sc0, sc1, sc2, sc3, sc4, sc5)
    sem_g = (sg0, sg1, sg2, sg3, sg4, sg5)
    sem_o = (so0, so1, so2, so3, so4, so5)

    pos_cp = pltpu.async_copy(pos_hbm.at[pl.ds(p0, _PBAND), :], rows0, sc0)
    x_cp = pltpu.async_copy(x_hbm.at[pl.ds(b0, _BB), pl.ds(p0, _PBAND)],
                            idx_v, sx)
    pltpu.sync_copy(seg_hbm, seg_v)
    segs = []
    for c in range(_D // _L):
        s0 = seg_v[0, pl.ds(c * _L, _L)]
        s1 = seg_v[1, pl.ds(c * _L, _L)]
        segs.append(jnp.where(band >= _NBAND // 2, s1, s0))
    pos_cp.wait()

    def _seg_add(r, carry):
        for c in range(_D // _L):
            sl = pl.ds(c * _L, _L)
            rows0[r, sl] = rows0[r, sl] + segs[c]
        return carry

    lax.fori_loop(0, _PBAND, _seg_add, 0)
    pltpu.sync_copy(rows0, comb_sp.at[pl.ds(sid * _PBAND, _PBAND), :])
    x_cp.wait()

    _K = len(rows)

    def comb_load(b):
        return pltpu.async_copy(comb_sp.at[pl.ds(sid * _PBAND, _PBAND), :],
                                rows[b % _K], sem_c[b % _K])

    def gather(b):
        return pltpu.async_copy(tok_hbm.at[idx_v.at[b]], rows[b % _K],
                                sem_g[b % _K], add=True)

    def out_store(b):
        return pltpu.async_copy(
            rows[b % _K],
            out_hbm.at[pl.ds((b0 + b) * _SEQ + p0, _PBAND), :], sem_o[b % _K])

    cps = {}
    for b in range(_K - 1):
        cps["c", b] = comb_load(b)
    for b in range(_BB):
        cps["c", b].wait()
        cps["g", b] = gather(b)
        if b >= 1:
            cps["g", b - 1].wait()
            cps["o", b - 1] = out_store(b - 1)
        if b + _K - 1 < _BB:
            if b >= 1:
                cps["o", b - 1].wait()
            cps["c", b + _K - 1] = comb_load(b + _K - 1)
    cps["g", _BB - 1].wait()
    cps["o", _BB - 1] = out_store(_BB - 1)
    for b in range(max(0, _BB - _K), _BB):
        cps["o", b].wait()


def kernel(x, token_table, pos_table, seg_table):
    out = _sc_embed(x.astype(jnp.int32), token_table, pos_table, seg_table)
    return out.reshape(_B, _SEQ, _D)


# speedup vs baseline: 1.0060x; 1.0060x over previous
import functools

import jax
import jax.numpy as jnp
from jax import lax
from jax.experimental import pallas as pl
from jax.experimental.pallas import tpu as pltpu
from jax.experimental.pallas import tpu_sc as plsc

_B = 16
_SEQ = 2048
_HALF = 1024
_D = 128
_ROWS = _B * _SEQ
_NC = 2
_NS = 16
_NW = _NC * _NS
_PER_W = _ROWS // _NW
_CHUNK = 128
_NCHUNK = _PER_W // _CHUNK
_L = 16

_sc_mesh = plsc.VectorSubcoreMesh(core_axis_name="c", subcore_axis_name="s")


@functools.partial(
    pl.kernel,
    out_type=jax.ShapeDtypeStruct((_ROWS, _D), jnp.float32),
    mesh=_sc_mesh,
    scratch_types=[
        pltpu.VMEM((_NCHUNK, _CHUNK), jnp.int32),
        pltpu.VMEM((2, _D), jnp.float32),
        pltpu.VMEM((_CHUNK, _D), jnp.float32),
        pltpu.VMEM((_CHUNK, _D), jnp.float32),
        pltpu.VMEM((_CHUNK, _D), jnp.float32),
        pltpu.VMEM((_CHUNK, _D), jnp.float32),
        pltpu.VMEM_SHARED((_SEQ, _D), jnp.float32),
        pltpu.SemaphoreType.DMA,
        pltpu.SemaphoreType.DMA,
        pltpu.SemaphoreType.DMA,
        pltpu.SemaphoreType.DMA,
        pltpu.SemaphoreType.DMA,
        pltpu.SemaphoreType.DMA,
        pltpu.SemaphoreType.DMA,
        pltpu.SemaphoreType.DMA,
        pltpu.SemaphoreType.DMA,
        pltpu.SemaphoreType.DMA,
        pltpu.SemaphoreType.DMA,
        pltpu.SemaphoreType.DMA,
    ],
)
def _sc_embed(x_hbm, tok_hbm, pos_hbm, seg_hbm, out_hbm,
              idx_v, seg_v, rows0, rows1, rows2, rows3, comb_sp,
              sc0, sc1, sc2, sc3, sg0, sg1, sg2, sg3, so0, so1, so2, so3):
    sid = lax.axis_index("s")
    wid = sid * _NC + lax.axis_index("c")
    base = wid * _PER_W
    pos0 = (wid % 2) * _HALF
    rows = (rows0, rows1, rows2, rows3)
    sem_c = (sc0, sc1, sc2, sc3)
    sem_g = (sg0, sg1, sg2, sg3)
    sem_o = (so0, so1, so2, so3)

    pos_cp = pltpu.async_copy(pos_hbm.at[pl.ds(sid * _CHUNK, _CHUNK), :],
                              rows0, sc0)
    x_cp = pltpu.async_copy(x_hbm.at[pl.ds(wid * _NCHUNK, _NCHUNK), :],
                            idx_v, sc1)
    pltpu.sync_copy(seg_hbm, seg_v)
    segs = []
    for c in range(_D // _L):
        s0 = seg_v[0, pl.ds(c * _L, _L)]
        s1 = seg_v[1, pl.ds(c * _L, _L)]
        segs.append(jnp.where(sid >= _NS // 2, s1, s0))
    pos_cp.wait()

    def _seg_add(r, carry):
        for c in range(_D // _L):
            sl = pl.ds(c * _L, _L)
            rows0[r, sl] = rows0[r, sl] + segs[c]
        return carry

    lax.fori_loop(0, _CHUNK, _seg_add, 0)
    pltpu.sync_copy(rows0, comb_sp.at[pl.ds(sid * _CHUNK, _CHUNK), :])
    x_cp.wait()
    plsc.subcore_barrier()

    def comb_load(j):
        return pltpu.async_copy(
            comb_sp.at[pl.ds(pos0 + j * _CHUNK, _CHUNK), :],
            rows[j % 4], sem_c[j % 4])

    def gather(j):
        return pltpu.async_copy(tok_hbm.at[idx_v.at[j]], rows[j % 4],
                                sem_g[j % 4], add=True)

    def out_store(j):
        return pltpu.async_copy(
            rows[j % 4], out_hbm.at[pl.ds(base + j * _CHUNK, _CHUNK), :],
            sem_o[j % 4])

    cps = {}
    for j in range(3):
        cps["c", j] = comb_load(j)
    for j in range(_NCHUNK):
        cps["c", j].wait()
        cps["g", j] = gather(j)
        if j >= 1:
            cps["g", j - 1].wait()
            cps["o", j - 1] = out_store(j - 1)
        if j + 3 < _NCHUNK:
            if j >= 1:
                cps["o", j - 1].wait()
            cps["c", j + 3] = comb_load(j + 3)
    cps["g", _NCHUNK - 1].wait()
    cps["o", _NCHUNK - 1] = out_store(_NCHUNK - 1)
    for j in range(4, _NCHUNK):
        cps["o", j].wait()


def kernel(x, token_table, pos_table, seg_table):
    x2d = x.reshape(_ROWS // _CHUNK, _CHUNK).astype(jnp.int32)
    out = _sc_embed(x2d, token_table, pos_table, seg_table)
    return out.reshape(_B, _SEQ, _D)
